# baseline (speedup 1.0000x reference)
import functools

import jax
import jax.numpy as jnp
import numpy as np
from jax import lax
from jax.experimental import pallas as pl
from jax.experimental.pallas import tpu as pltpu
from jax.experimental.pallas import tpu_sc as plsc

N = 10000
E = 320000
D = 128
DE = 16
EPS = 0.1

NC, NS = 2, 16
NW = NC * NS
EPW = E // NW
K = 40
NCHUNK = EPW // K
NP = 10112
RPT = NP // NS
VEC = 16


def _sc_segment(pre, eaw, ei4):
    mesh = plsc.VectorSubcoreMesh(core_axis_name="c", subcore_axis_name="s")

    @functools.partial(
        pl.kernel,
        out_type=jax.ShapeDtypeStruct((NC, NP, D), jnp.float32),
        mesh=mesh,
        scratch_types=[
            pltpu.VMEM((K,), jnp.int32),
            pltpu.VMEM((K,), jnp.int32),
            pltpu.VMEM((K,), jnp.int32),
            pltpu.VMEM((K,), jnp.int32),
            pltpu.VMEM((K,), jnp.int32),
            pltpu.VMEM((K,), jnp.int32),
            pltpu.VMEM((K, D), jnp.float32),
            pltpu.VMEM((K, D), jnp.float32),
            pltpu.VMEM((K, D), jnp.float32),
            pltpu.VMEM((K, D), jnp.float32),
            pltpu.VMEM((K, D), jnp.float32),
            pltpu.VMEM((K, D), jnp.float32),
            pltpu.VMEM_SHARED((NP, D), jnp.float32),
            pltpu.SemaphoreType.DMA,
            pltpu.SemaphoreType.DMA,
            pltpu.SemaphoreType.DMA,
            pltpu.SemaphoreType.DMA,
            pltpu.SemaphoreType.DMA,
            pltpu.SemaphoreType.DMA,
            pltpu.SemaphoreType.DMA,
            pltpu.SemaphoreType.DMA,
            pltpu.SemaphoreType.DMA,
            pltpu.SemaphoreType.DMA,
            pltpu.SemaphoreType.DMA,
            pltpu.SemaphoreType.DMA,
        ],
    )
    def k(pre_h, eaw_h, ei_h, s_out,
          sb0, sb1, db0, db1, db2, db3, xr0, xr1, er0, er1, mr0, mr1, s_sh,
          sg0, sg1, se0, se1, si0, si1, sd0, sd1, sd2, sd3, sc0, sc1):
        c = lax.axis_index("c")
        t = lax.axis_index("s")
        wid = c * NS + t
        r0 = t * RPT
        sb = (sb0, sb1)
        db = (db0, db1, db2, db3)
        xr = (xr0, xr1)
        er = (er0, er1)
        mr = (mr0, mr1)
        sg = (sg0, sg1)
        se = (se0, se1)
        si = (si0, si1)
        sd = (sd0, sd1, sd2, sd3)
        sc = (sc0, sc1)

        base_w = wid * EPW

        def issue_ge(ci, b):
            pltpu.async_copy(pre_h.at[sb[b]], xr[b], sg[b])
            pltpu.async_copy(eaw_h.at[pl.ds(base_w + ci * K, K)], er[b], se[b])

        def wait_ge(ci, b):
            pltpu.make_async_copy(pre_h.at[sb[b]], xr[b], sg[b]).wait()
            pltpu.make_async_copy(
                eaw_h.at[pl.ds(base_w + ci * K, K)], er[b], se[b]).wait()

        def process(ci, b, p):
            wait_ge(ci, b)

            @pl.when(ci >= 2)
            def _():
                pltpu.make_async_copy(
                    mr[b], s_sh.at[db[p]], sc[b]).wait()

            @pl.when(ci + 2 < NCHUNK)
            def _():
                pltpu.async_copy(ei_h.at[0, wid, ci + 2], sb[b], si[b])
                pltpu.async_copy(
                    ei_h.at[1, wid, ci + 2], db[(p + 2) % 4], sd[(p + 2) % 4])

            def row(i):
                for j in range(D // VEC):
                    sl = pl.ds(j * VEC, VEC)
                    mr[b][i, sl] = jnp.maximum(
                        xr[b][i, sl] + er[b][i, sl], 0.0)

            plsc.parallel_loop(0, K, 1, unroll=2)(row)

            @pl.when(ci >= 2)
            def _():
                pltpu.make_async_copy(ei_h.at[1, wid, ci], db[p], sd[p]).wait()

            pltpu.async_copy(mr[b], s_sh.at[db[p]], sc[b], add=True)

            @pl.when(ci + 2 < NCHUNK)
            def _():
                pltpu.make_async_copy(ei_h.at[0, wid, ci + 2], sb[b], si[b]).wait()
                issue_ge(ci + 2, b)

        pltpu.sync_copy(ei_h.at[0, wid, 0], sb0)
        pltpu.sync_copy(ei_h.at[0, wid, 1], sb1)
        pltpu.sync_copy(ei_h.at[1, wid, 0], db0)
        pltpu.sync_copy(ei_h.at[1, wid, 1], db1)
        issue_ge(0, 0)
        issue_ge(1, 1)

        def zr(i, carry):
            for j in range(D // VEC):
                mr0[i, pl.ds(j * VEC, VEC)] = jnp.zeros((VEC,), jnp.float32)
            return carry

        lax.fori_loop(0, K, zr, 0)
        for r in range(RPT // K):
            pltpu.sync_copy(mr0, s_sh.at[pl.ds(r0 + r * K, K)])
        if RPT % K:
            pltpu.sync_copy(mr0.at[pl.ds(0, RPT % K)],
                            s_sh.at[pl.ds(r0 + (RPT // K) * K, RPT % K)])
        plsc.subcore_barrier()

        def quad(q, carry):
            ci = q * 4
            process(ci, 0, 0)
            process(ci + 1, 1, 1)
            process(ci + 2, 0, 2)
            process(ci + 3, 1, 3)
            return carry

        lax.fori_loop(0, NCHUNK // 4, quad, 0)
        for ci in range(NCHUNK - NCHUNK % 4, NCHUNK):
            process(jnp.int32(ci), ci % 2, ci % 4)

        for b in (0, 1):
            pltpu.make_async_copy(mr[b], s_sh.at[db[b]], sc[b]).wait()

        plsc.subcore_barrier()
        pltpu.sync_copy(s_sh.at[pl.ds(r0, RPT)], s_out.at[c, pl.ds(r0, RPT)])

    return k(pre, eaw, ei4)


def _sc_counts(ei4, zcnt, ones):
    mesh = plsc.VectorSubcoreMesh(core_axis_name="c", subcore_axis_name="s")

    @functools.partial(
        pl.kernel,
        out_type=jax.ShapeDtypeStruct((NC, NP), jnp.float32),
        mesh=mesh,
        scratch_types=[
            pltpu.VMEM((NCHUNK, K), jnp.int32),
            pltpu.VMEM((K,), jnp.float32),
            pltpu.VMEM_SHARED((NP,), jnp.float32),
        ],
    )
    def k(ei_h, zcnt_h, ones_h, cnt_out, idx_v, ones_v, cnt_sh):
        c = lax.axis_index("c")
        t = lax.axis_index("s")
        wid = c * NS + t

        @pl.when(t == 0)
        def _():
            pltpu.sync_copy(zcnt_h, cnt_sh)

        pltpu.sync_copy(ei_h.at[1, wid], idx_v)
        pltpu.sync_copy(ones_h, ones_v)
        plsc.subcore_barrier()

        def chunk(ci, carry):
            pltpu.sync_copy(ones_v, cnt_sh.at[idx_v.at[ci]], add=True)
            return carry

        lax.fori_loop(0, NCHUNK, chunk, 0)
        plsc.subcore_barrier()

        @pl.when(t == 0)
        def _():
            pltpu.sync_copy(cnt_sh, cnt_out.at[c])

    return k(ei4, zcnt, ones)


BE = 12800
RB = 2000


def _tc_eaw(ea, w):
    def body(ea_ref, w_ref, o_ref):
        o_ref[...] = jnp.dot(ea_ref[...], w_ref[...])

    return pl.pallas_call(
        body,
        grid=(E // BE,),
        in_specs=[pl.BlockSpec((BE, DE), lambda i: (i, 0)),
                  pl.BlockSpec((DE, D), lambda i: (0, 0))],
        out_specs=pl.BlockSpec((BE, D), lambda i: (i, 0)),
        out_shape=jax.ShapeDtypeStruct((E, D), jnp.float32),
    )(ea, w)


def _tc_entry(h, wmx, bm, wax):
    def body(h_ref, wm_ref, bm_ref, wa_ref, pre_ref, hw_ref):
        hh = h_ref[...]
        pre_ref[...] = jnp.dot(hh, wm_ref[...]) + bm_ref[...]
        hw_ref[...] = jnp.dot(hh, wa_ref[...])

    out = jax.ShapeDtypeStruct((N, D), jnp.float32)
    nspec = pl.BlockSpec((RB, D), lambda i: (i, 0))
    wspec = pl.BlockSpec((D, D), lambda i: (0, 0))
    bspec = pl.BlockSpec((1, D), lambda i: (0, 0))
    return pl.pallas_call(
        body,
        grid=(N // RB,),
        in_specs=[nspec, wspec, bspec, wspec],
        out_specs=[nspec, nspec],
        out_shape=[out, out],
    )(h, wmx, bm, wax)


def _tc_post_mid(s, cnt, hw, origin, waa, ba, wmx_n, bm_n, wax_n):
    def body(s_ref, c_ref, hw_ref, org_ref, waa_ref, ba_ref,
             wmn_ref, bmn_ref, wan_ref, h_ref, pre_ref, hwn_ref):
        sm = s_ref[0] + s_ref[1]
        cc = jnp.maximum(c_ref[0] + c_ref[1], 1.0)
        agg = sm / cc
        t = jnp.maximum(
            jnp.dot(agg, waa_ref[...]) + hw_ref[...] + ba_ref[...],
            0.0)
        h = org_ref[...] + EPS * t
        h_ref[...] = h
        pre_ref[...] = jnp.dot(h, wmn_ref[...]) + bmn_ref[...]
        hwn_ref[...] = jnp.dot(h, wan_ref[...])

    out = jax.ShapeDtypeStruct((N, D), jnp.float32)
    nspec = pl.BlockSpec((RB, D), lambda i: (i, 0))
    wspec = pl.BlockSpec((D, D), lambda i: (0, 0))
    bspec = pl.BlockSpec((1, D), lambda i: (0, 0))
    return pl.pallas_call(
        body,
        grid=(N // RB,),
        in_specs=[
            pl.BlockSpec((NC, RB, D), lambda i: (0, i, 0)),
            pl.BlockSpec((NC, RB, 1), lambda i: (0, i, 0)),
            nspec, nspec, wspec, bspec, wspec, bspec, wspec,
        ],
        out_specs=[nspec, nspec, nspec],
        out_shape=[out, out, out],
    )(s, cnt, hw, origin, waa, ba, wmx_n, bm_n, wax_n)


def _tc_post_last(s, cnt, hw, waa, ba):
    def body(s_ref, c_ref, hw_ref, waa_ref, ba_ref, h_ref):
        sm = s_ref[0] + s_ref[1]
        cc = jnp.maximum(c_ref[0] + c_ref[1], 1.0)
        agg = sm / cc
        h_ref[...] = jnp.maximum(
            jnp.dot(agg, waa_ref[...]) + hw_ref[...] + ba_ref[...],
            0.0)

    out = jax.ShapeDtypeStruct((N, D), jnp.float32)
    nspec = pl.BlockSpec((RB, D), lambda i: (i, 0))
    wspec = pl.BlockSpec((D, D), lambda i: (0, 0))
    bspec = pl.BlockSpec((1, D), lambda i: (0, 0))
    return pl.pallas_call(
        body,
        grid=(N // RB,),
        in_specs=[
            pl.BlockSpec((NC, RB, D), lambda i: (0, i, 0)),
            pl.BlockSpec((NC, RB, 1), lambda i: (0, i, 0)),
            nspec, wspec, bspec,
        ],
        out_specs=nspec,
        out_shape=out,
    )(s, cnt, hw, waa, ba)


def kernel(x, edge_attr, edge_index, Wm0, bm0, Wa0, ba0,
           Wm1, bm1, Wa1, ba1, Wm2, bm2, Wa2, ba2):
    x0 = x[0]
    ea = edge_attr[0]
    ei4 = edge_index[0].reshape(2, NW, NCHUNK, K)

    zcnt = jnp.zeros((NP,), jnp.float32)
    ones = jnp.ones((K,), jnp.float32)

    Ws = [(Wm0, bm0, Wa0, ba0), (Wm1, bm1, Wa1, ba1), (Wm2, bm2, Wa2, ba2)]
    eaw0 = _tc_eaw(ea, Wm0[D:])
    cnt = _sc_counts(ei4, zcnt, ones)
    cnt3 = cnt.reshape(NC, NP, 1)

    pre, hw = _tc_entry(x0, Wm0[:D], bm0.reshape(1, D), Wa0[D:])
    eaw = [eaw0, None, None]
    h = None
    for i in range(3):
        _, _, Wa, ba = Ws[i]
        s = _sc_segment(pre, eaw[i], ei4)
        if i < 2:
            eaw[i + 1] = _tc_eaw(ea, Ws[i + 1][0][D:])
        if i < 2:
            Wm_n, bm_n, Wa_n, _ = Ws[i + 1]
            h, pre, hw = _tc_post_mid(
                s, cnt3, hw, x0, Wa[:D],
                ba.reshape(1, D), Wm_n[:D], bm_n.reshape(1, D), Wa_n[D:])
        else:
            h = _tc_post_last(s, cnt3, hw, Wa[:D], ba.reshape(1, D))
    return h[None]


# speedup vs baseline: 1.0017x; 1.0017x over previous
import functools

import jax
import jax.numpy as jnp
from jax import lax
from jax.experimental import pallas as pl
from jax.experimental.pallas import tpu as pltpu
from jax.experimental.pallas import tpu_sc as plsc

N = 10000
E = 320000
D = 128
DE = 16
EPS = 0.1

NC, NS = 2, 16
NW = NC * NS
EPW = E // NW
K = 40
NCHUNK = EPW // K
NP = 10112
RPT = NP // NS
VEC = 16


def _sc_segment(pre, eaw, ei4):
    mesh = plsc.VectorSubcoreMesh(core_axis_name="c", subcore_axis_name="s")

    @functools.partial(
        pl.kernel,
        out_type=jax.ShapeDtypeStruct((NC, NP, D), jnp.float32),
        mesh=mesh,
        scratch_types=[
            pltpu.VMEM((K,), jnp.int32),
            pltpu.VMEM((K,), jnp.int32),
            pltpu.VMEM((K,), jnp.int32),
            pltpu.VMEM((K,), jnp.int32),
            pltpu.VMEM((K,), jnp.int32),
            pltpu.VMEM((K,), jnp.int32),
            pltpu.VMEM((K, D), jnp.float32),
            pltpu.VMEM((K, D), jnp.float32),
            pltpu.VMEM((K, D), jnp.float32),
            pltpu.VMEM((K, D), jnp.float32),
            pltpu.VMEM((K, D), jnp.float32),
            pltpu.VMEM((K, D), jnp.float32),
            pltpu.VMEM_SHARED((NP, D), jnp.float32),
            pltpu.SemaphoreType.DMA,
            pltpu.SemaphoreType.DMA,
            pltpu.SemaphoreType.DMA,
            pltpu.SemaphoreType.DMA,
            pltpu.SemaphoreType.DMA,
            pltpu.SemaphoreType.DMA,
            pltpu.SemaphoreType.DMA,
            pltpu.SemaphoreType.DMA,
            pltpu.SemaphoreType.DMA,
            pltpu.SemaphoreType.DMA,
            pltpu.SemaphoreType.DMA,
            pltpu.SemaphoreType.DMA,
        ],
    )
    def k(pre_h, eaw_h, ei_h, s_out,
          sb0, sb1, db0, db1, db2, db3, xr0, xr1, er0, er1, mr0, mr1, s_sh,
          sg0, sg1, se0, se1, si0, si1, sd0, sd1, sd2, sd3, sc0, sc1):
        c = lax.axis_index("c")
        t = lax.axis_index("s")
        wid = c * NS + t
        r0 = t * RPT
        sb = (sb0, sb1)
        db = (db0, db1, db2, db3)
        xr = (xr0, xr1)
        er = (er0, er1)
        mr = (mr0, mr1)
        sg = (sg0, sg1)
        se = (se0, se1)
        si = (si0, si1)
        sd = (sd0, sd1, sd2, sd3)
        sc = (sc0, sc1)

        base_w = wid * EPW

        def issue_ge(ci, b):
            pltpu.async_copy(pre_h.at[sb[b]], xr[b], sg[b])
            pltpu.async_copy(eaw_h.at[pl.ds(base_w + ci * K, K)], er[b], se[b])

        def wait_ge(ci, b):
            pltpu.make_async_copy(pre_h.at[sb[b]], xr[b], sg[b]).wait()
            pltpu.make_async_copy(
                eaw_h.at[pl.ds(base_w + ci * K, K)], er[b], se[b]).wait()

        def process(ci, b, p):
            wait_ge(ci, b)

            @pl.when(ci >= 2)
            def _():
                pltpu.make_async_copy(
                    mr[b], s_sh.at[db[p]], sc[b]).wait()

            @pl.when(ci + 2 < NCHUNK)
            def _():
                pltpu.async_copy(ei_h.at[0, wid, ci + 2], sb[b], si[b])
                pltpu.async_copy(
                    ei_h.at[1, wid, ci + 2], db[(p + 2) % 4], sd[(p + 2) % 4])

            def row(i):
                for j in range(D // VEC):
                    sl = pl.ds(j * VEC, VEC)
                    mr[b][i, sl] = jnp.maximum(
                        xr[b][i, sl] + er[b][i, sl], 0.0)

            plsc.parallel_loop(0, K, 1, unroll=2)(row)

            @pl.when(ci >= 2)
            def _():
                pltpu.make_async_copy(ei_h.at[1, wid, ci], db[p], sd[p]).wait()

            pltpu.async_copy(mr[b], s_sh.at[db[p]], sc[b], add=True)

            @pl.when(ci + 2 < NCHUNK)
            def _():
                pltpu.make_async_copy(ei_h.at[0, wid, ci + 2], sb[b], si[b]).wait()
                issue_ge(ci + 2, b)

        pltpu.sync_copy(ei_h.at[0, wid, 0], sb0)
        pltpu.sync_copy(ei_h.at[0, wid, 1], sb1)
        pltpu.sync_copy(ei_h.at[1, wid, 0], db0)
        pltpu.sync_copy(ei_h.at[1, wid, 1], db1)
        issue_ge(0, 0)
        issue_ge(1, 1)

        def zr(i, carry):
            for j in range(D // VEC):
                mr0[i, pl.ds(j * VEC, VEC)] = jnp.zeros((VEC,), jnp.float32)
            return carry

        lax.fori_loop(0, K, zr, 0)
        for r in range(RPT // K):
            pltpu.sync_copy(mr0, s_sh.at[pl.ds(r0 + r * K, K)])
        if RPT % K:
            pltpu.sync_copy(mr0.at[pl.ds(0, RPT % K)],
                            s_sh.at[pl.ds(r0 + (RPT // K) * K, RPT % K)])
        plsc.subcore_barrier()

        def quad(q, carry):
            ci = q * 4
            process(ci, 0, 0)
            process(ci + 1, 1, 1)
            process(ci + 2, 0, 2)
            process(ci + 3, 1, 3)
            return carry

        lax.fori_loop(0, NCHUNK // 4, quad, 0)
        for ci in range(NCHUNK - NCHUNK % 4, NCHUNK):
            process(jnp.int32(ci), ci % 2, ci % 4)

        for b in (0, 1):
            pltpu.make_async_copy(mr[b], s_sh.at[db[b]], sc[b]).wait()

        plsc.subcore_barrier()
        pltpu.sync_copy(s_sh.at[pl.ds(r0, RPT)], s_out.at[c, pl.ds(r0, RPT)])

    return k(pre, eaw, ei4)


def _sc_counts(ei4, zcnt, ones):
    mesh = plsc.VectorSubcoreMesh(core_axis_name="c", subcore_axis_name="s")

    @functools.partial(
        pl.kernel,
        out_type=jax.ShapeDtypeStruct((NC, NP), jnp.float32),
        mesh=mesh,
        scratch_types=[
            pltpu.VMEM((NCHUNK, K), jnp.int32),
            pltpu.VMEM((K,), jnp.float32),
            pltpu.VMEM_SHARED((NP,), jnp.float32),
        ],
    )
    def k(ei_h, zcnt_h, ones_h, cnt_out, idx_v, ones_v, cnt_sh):
        c = lax.axis_index("c")
        t = lax.axis_index("s")
        wid = c * NS + t

        @pl.when(t == 0)
        def _():
            pltpu.sync_copy(zcnt_h, cnt_sh)

        pltpu.sync_copy(ei_h.at[1, wid], idx_v)
        pltpu.sync_copy(ones_h, ones_v)
        plsc.subcore_barrier()

        def chunk(ci, carry):
            pltpu.sync_copy(ones_v, cnt_sh.at[idx_v.at[ci]], add=True)
            return carry

        lax.fori_loop(0, NCHUNK, chunk, 0)
        plsc.subcore_barrier()

        @pl.when(t == 0)
        def _():
            pltpu.sync_copy(cnt_sh, cnt_out.at[c])

    return k(ei4, zcnt, ones)


BE = 6400
RB = 2000


def _tc_eaw(ea, w):
    def body(ea_ref, w_ref, o_ref):
        o_ref[...] = jnp.dot(ea_ref[...], w_ref[...])

    return pl.pallas_call(
        body,
        grid=(E // BE,),
        in_specs=[pl.BlockSpec((BE, DE), lambda i: (i, 0)),
                  pl.BlockSpec((DE, D), lambda i: (0, 0))],
        out_specs=pl.BlockSpec((BE, D), lambda i: (i, 0)),
        out_shape=jax.ShapeDtypeStruct((E, D), jnp.float32),
    )(ea, w)


def _tc_entry(h, wmx, bm, wax):
    def body(h_ref, wm_ref, bm_ref, wa_ref, pre_ref, hw_ref):
        hh = h_ref[...]
        pre_ref[...] = jnp.dot(hh, wm_ref[...]) + bm_ref[...]
        hw_ref[...] = jnp.dot(hh, wa_ref[...])

    out = jax.ShapeDtypeStruct((N, D), jnp.float32)
    nspec = pl.BlockSpec((RB, D), lambda i: (i, 0))
    wspec = pl.BlockSpec((D, D), lambda i: (0, 0))
    bspec = pl.BlockSpec((1, D), lambda i: (0, 0))
    return pl.pallas_call(
        body,
        grid=(N // RB,),
        in_specs=[nspec, wspec, bspec, wspec],
        out_specs=[nspec, nspec],
        out_shape=[out, out],
    )(h, wmx, bm, wax)


def _tc_post_mid(s, cnt, hw, origin, waa, ba, wmx_n, bm_n, wax_n):
    def body(s_ref, c_ref, hw_ref, org_ref, waa_ref, ba_ref,
             wmn_ref, bmn_ref, wan_ref, h_ref, pre_ref, hwn_ref):
        sm = s_ref[0] + s_ref[1]
        cc = jnp.maximum(c_ref[0] + c_ref[1], 1.0)
        agg = sm / cc
        t = jnp.maximum(
            jnp.dot(agg, waa_ref[...]) + hw_ref[...] + ba_ref[...],
            0.0)
        h = org_ref[...] + EPS * t
        h_ref[...] = h
        pre_ref[...] = jnp.dot(h, wmn_ref[...]) + bmn_ref[...]
        hwn_ref[...] = jnp.dot(h, wan_ref[...])

    out = jax.ShapeDtypeStruct((N, D), jnp.float32)
    nspec = pl.BlockSpec((RB, D), lambda i: (i, 0))
    wspec = pl.BlockSpec((D, D), lambda i: (0, 0))
    bspec = pl.BlockSpec((1, D), lambda i: (0, 0))
    return pl.pallas_call(
        body,
        grid=(N // RB,),
        in_specs=[
            pl.BlockSpec((NC, RB, D), lambda i: (0, i, 0)),
            pl.BlockSpec((NC, RB, 1), lambda i: (0, i, 0)),
            nspec, nspec, wspec, bspec, wspec, bspec, wspec,
        ],
        out_specs=[nspec, nspec, nspec],
        out_shape=[out, out, out],
    )(s, cnt, hw, origin, waa, ba, wmx_n, bm_n, wax_n)


def _tc_post_last(s, cnt, hw, waa, ba):
    def body(s_ref, c_ref, hw_ref, waa_ref, ba_ref, h_ref):
        sm = s_ref[0] + s_ref[1]
        cc = jnp.maximum(c_ref[0] + c_ref[1], 1.0)
        agg = sm / cc
        h_ref[...] = jnp.maximum(
            jnp.dot(agg, waa_ref[...]) + hw_ref[...] + ba_ref[...],
            0.0)

    out = jax.ShapeDtypeStruct((N, D), jnp.float32)
    nspec = pl.BlockSpec((RB, D), lambda i: (i, 0))
    wspec = pl.BlockSpec((D, D), lambda i: (0, 0))
    bspec = pl.BlockSpec((1, D), lambda i: (0, 0))
    return pl.pallas_call(
        body,
        grid=(N // RB,),
        in_specs=[
            pl.BlockSpec((NC, RB, D), lambda i: (0, i, 0)),
            pl.BlockSpec((NC, RB, 1), lambda i: (0, i, 0)),
            nspec, wspec, bspec,
        ],
        out_specs=nspec,
        out_shape=out,
    )(s, cnt, hw, waa, ba)


def kernel(x, edge_attr, edge_index, Wm0, bm0, Wa0, ba0,
           Wm1, bm1, Wa1, ba1, Wm2, bm2, Wa2, ba2):
    x0 = x[0]
    ea = edge_attr[0]
    ei4 = edge_index[0].reshape(2, NW, NCHUNK, K)

    zcnt = jnp.zeros((NP,), jnp.float32)
    ones = jnp.ones((K,), jnp.float32)

    Ws = [(Wm0, bm0, Wa0, ba0), (Wm1, bm1, Wa1, ba1), (Wm2, bm2, Wa2, ba2)]
    eaw0 = _tc_eaw(ea, Wm0[D:])
    cnt = _sc_counts(ei4, zcnt, ones)
    cnt3 = cnt.reshape(NC, NP, 1)

    pre, hw = _tc_entry(x0, Wm0[:D], bm0.reshape(1, D), Wa0[D:])
    eaw = [eaw0, None, None]
    h = None
    for i in range(3):
        _, _, Wa, ba = Ws[i]
        s = _sc_segment(pre, eaw[i], ei4)
        if i < 2:
            eaw[i + 1] = _tc_eaw(ea, Ws[i + 1][0][D:])
        if i < 2:
            Wm_n, bm_n, Wa_n, _ = Ws[i + 1]
            h, pre, hw = _tc_post_mid(
                s, cnt3, hw, x0, Wa[:D],
                ba.reshape(1, D), Wm_n[:D], bm_n.reshape(1, D), Wa_n[D:])
        else:
            h = _tc_post_last(s, cnt3, hw, Wa[:D], ba.reshape(1, D))
    return h[None]
